# baseline (speedup 1.0000x reference)
import functools

import jax
import jax.numpy as jnp
from jax import lax
from jax.experimental import pallas as pl
from jax.experimental.pallas import tpu as pltpu
from jax.experimental.pallas import tpu_sc as plsc

_V = 100000
_E = 64
_EP = 128
_H = 256
_C = 20
_B = 1024

_NC = 2
_NS = 16
_NW = _NC * _NS
_ROWS = _B * _C
_RPW = _ROWS // _NW

_TV = 3072
_NVT = (_V + _TV - 1) // _TV


_TT = 2048


def _transpose_pad_body(in_ref, out_ref):
  t = jnp.transpose(in_ref[...], (1, 0))
  out_ref[:, : _E] = t
  out_ref[:, _E:] = jnp.zeros((_TT, _EP - _E), jnp.float32)


def _transpose_pad(emb_t):
  return pl.pallas_call(
      _transpose_pad_body,
      grid=((_V + _TT - 1) // _TT,),
      in_specs=[pl.BlockSpec((_E, _TT), lambda j: (0, j))],
      out_specs=pl.BlockSpec((_TT, _EP), lambda j: (j, 0)),
      out_shape=jax.ShapeDtypeStruct((_V, _EP), jnp.float32),
  )(emb_t)


def _sc_gather(table, idx):
  mesh = plsc.VectorSubcoreMesh(core_axis_name="c", subcore_axis_name="s")

  @functools.partial(
      pl.kernel,
      mesh=mesh,
      out_type=jax.ShapeDtypeStruct((_ROWS, _EP), jnp.float32),
      scratch_types=[
          pltpu.VMEM((_RPW,), jnp.int32),
          pltpu.VMEM((_RPW, _EP), jnp.float32),
          pltpu.SemaphoreType.DMA,
      ],
  )
  def gather_kernel(table_hbm, idx_hbm, out_hbm, idx_v, rows_v, sem):
    wid = lax.axis_index("s") * _NC + lax.axis_index("c")
    base = wid * _RPW
    pltpu.sync_copy(idx_hbm.at[pl.ds(base, _RPW)], idx_v)
    pltpu.async_copy(table_hbm.at[idx_v], rows_v, sem).wait()
    pltpu.sync_copy(rows_v, out_hbm.at[pl.ds(base, _RPW)])

  return gather_kernel(table, idx)


def _mlp_body(flat_ref, w1_ref, b1_ref, w2t_ref, b2_ref, out_ref, hid_ref):
  @pl.when(pl.program_id(0) == 0)
  def _():
    ht = lax.dot_general(
        w1_ref[...], flat_ref[...],
        dimension_numbers=(((0,), (1,)), ((), ())),
        precision=lax.Precision.HIGHEST)
    hid_ref[...] = jnp.tanh(ht + b1_ref[...]).astype(jnp.bfloat16)

  w2t = w2t_ref[...].astype(jnp.bfloat16)
  acc = lax.dot(w2t, hid_ref[...], preferred_element_type=jnp.float32)
  out_ref[...] = acc + b2_ref[...].T


def kernel(x, embedding, W1, b1, W2, b2):
  idx = x.reshape(-1).astype(jnp.int32)
  table = _transpose_pad(embedding.T)
  rows = _sc_gather(table, idx)
  flat = rows.reshape(_B, _C * _EP)
  w1p = jnp.pad(
      W1.reshape(_C, _E, _H), ((0, 0), (0, _EP - _E), (0, 0))
  ).reshape(_C * _EP, _H)
  logits_t = pl.pallas_call(
      _mlp_body,
      grid=(_NVT,),
      in_specs=[
          pl.BlockSpec((_B, _C * _EP), lambda j: (0, 0)),
          pl.BlockSpec((_C * _EP, _H), lambda j: (0, 0)),
          pl.BlockSpec((_H, 1), lambda j: (0, 0)),
          pl.BlockSpec((_TV, _H), lambda j: (j, 0)),
          pl.BlockSpec((1, _TV), lambda j: (0, j)),
      ],
      out_specs=pl.BlockSpec((_TV, _B), lambda j: (j, 0)),
      out_shape=jax.ShapeDtypeStruct((_V, _B), jnp.float32),
      scratch_shapes=[pltpu.VMEM((_H, _B), jnp.bfloat16)],
  )(flat, w1p, b1.reshape(_H, 1), W2.T, b2.reshape(1, _V))
  return logits_t.T


# speedup vs baseline: 1.0818x; 1.0818x over previous
import functools

import jax
import jax.numpy as jnp
from jax import lax
from jax.experimental import pallas as pl
from jax.experimental.pallas import tpu as pltpu
from jax.experimental.pallas import tpu_sc as plsc

_V = 100000
_E = 64
_EP = 128
_H = 256
_C = 20
_B = 1024

_NC = 2
_NS = 16
_NW = _NC * _NS
_ROWS = _B * _C
_RPW = _ROWS // _NW

_TV = 5120
_NVT = (_V + _TV - 1) // _TV


_TT = 8192


def _transpose_pad_body(in_ref, out_ref):
  t = jnp.transpose(in_ref[...], (1, 0))
  out_ref[:, : _E] = t
  out_ref[:, _E:] = jnp.zeros((_TT, _EP - _E), jnp.float32)


def _transpose_pad(emb_t):
  return pl.pallas_call(
      _transpose_pad_body,
      grid=((_V + _TT - 1) // _TT,),
      in_specs=[pl.BlockSpec((_E, _TT), lambda j: (0, j))],
      out_specs=pl.BlockSpec((_TT, _EP), lambda j: (j, 0)),
      out_shape=jax.ShapeDtypeStruct((_V, _EP), jnp.float32),
  )(emb_t)


def _sc_gather(table, idx):
  mesh = plsc.VectorSubcoreMesh(core_axis_name="c", subcore_axis_name="s")

  @functools.partial(
      pl.kernel,
      mesh=mesh,
      out_type=jax.ShapeDtypeStruct((_ROWS, _EP), jnp.float32),
      scratch_types=[
          pltpu.VMEM((_RPW,), jnp.int32),
          pltpu.VMEM((_RPW, _EP), jnp.float32),
          pltpu.SemaphoreType.DMA,
      ],
  )
  def gather_kernel(table_hbm, idx_hbm, out_hbm, idx_v, rows_v, sem):
    wid = lax.axis_index("s") * _NC + lax.axis_index("c")
    base = wid * _RPW
    pltpu.sync_copy(idx_hbm.at[pl.ds(base, _RPW)], idx_v)
    pltpu.async_copy(table_hbm.at[idx_v], rows_v, sem).wait()
    pltpu.sync_copy(rows_v, out_hbm.at[pl.ds(base, _RPW)])

  return gather_kernel(table, idx)


def _hidden_body(flat_ref, w1_ref, b1_ref, hid_ref):
  ht = lax.dot_general(
      w1_ref[...], flat_ref[...],
      dimension_numbers=(((0,), (1,)), ((), ())),
      precision=lax.Precision.HIGHEST)
  hid_ref[...] = jnp.tanh(ht + b1_ref[...]).astype(jnp.bfloat16)


def _logits_body(hid_ref, w2t_ref, b2_ref, out_ref):
  w2t = w2t_ref[...].astype(jnp.bfloat16)
  acc = lax.dot(w2t, hid_ref[...], preferred_element_type=jnp.float32)
  out_ref[...] = acc + b2_ref[...].T


def kernel(x, embedding, W1, b1, W2, b2):
  idx = x.reshape(-1).astype(jnp.int32)
  table = _transpose_pad(embedding.T)
  rows = _sc_gather(table, idx)
  flat = rows.reshape(_B, _C * _EP)
  w1p = jnp.pad(
      W1.reshape(_C, _E, _H), ((0, 0), (0, _EP - _E), (0, 0))
  ).reshape(_C * _EP, _H)
  hid_t = pl.pallas_call(
      _hidden_body,
      out_shape=jax.ShapeDtypeStruct((_H, _B), jnp.bfloat16),
  )(flat, w1p, b1.reshape(_H, 1))
  logits_t = pl.pallas_call(
      _logits_body,
      grid=(_NVT,),
      in_specs=[
          pl.BlockSpec((_H, _B), lambda j: (0, 0)),
          pl.BlockSpec((_TV, _H), lambda j: (j, 0)),
          pl.BlockSpec((1, _TV), lambda j: (0, j)),
      ],
      out_specs=pl.BlockSpec((_TV, _B), lambda j: (j, 0)),
      out_shape=jax.ShapeDtypeStruct((_V, _B), jnp.float32),
  )(hid_t, W2.T, b2.reshape(1, _V))
  return logits_t.T
